# baseline (speedup 1.0000x reference)
import jax
import jax.numpy as jnp
from jax import lax
from jax.experimental import pallas as pl
from jax.experimental.pallas import tpu as pltpu
from jax.experimental.pallas import tpu_sc as plsc

_EPS = 1e-5
_CHUNK = 128
_LANES = 16
_TILES = 16
_CORES = 2

_SC_PARAMS = pltpu.CompilerParams(use_tc_tiling_on_sc=False)


def _row_block(n):
    return 1000 if n % 1000 == 0 else n


def _film_dense(xin, bn_pack, Wls, Wfsb, Wfsg, w_outs, b_outs, act):
    n, ci = xin.shape
    bn = _row_block(n)
    grid = (n // bn,)
    nouts = len(w_outs)
    has_bias = [b is not None for b in b_outs]

    def body(*refs):
        refs = list(refs)
        if bn_pack is not None:
            yr, csr, cqr, bwr, bbr = refs[:5]
            del refs[:5]
            mu = csr[...] / n
            var = cqr[...] / n - mu * mu
            xb = (yr[...] - mu) * lax.rsqrt(var + _EPS) * bwr[...] + bbr[...]
        else:
            xb = refs.pop(0)[...]
        wls, wfsb, wfsg = refs[0], refs[1], refs[2]
        wrefs = refs[3:3 + nouts]
        brefs = refs[3 + nouts:3 + nouts + sum(has_bias)]
        outs = refs[3 + nouts + sum(has_bias):]

        def dot(wr):
            return lax.dot_general(xb, wr[...], (((1,), (1,)), ((), ())),
                                   preferred_element_type=jnp.float32)

        skip = dot(wfsg) * dot(wls) + dot(wfsb)
        if act:
            skip = jnp.maximum(skip, 0.0)
        outs[0][...] = skip
        bi = 0
        for k in range(nouts):
            r = dot(wrefs[k])
            if has_bias[k]:
                r = r + brefs[bi][...]
                bi += 1
            outs[1 + k][...] = r

    args = [xin]
    in_specs = [pl.BlockSpec((bn, ci), lambda i: (i, 0))]
    if bn_pack is not None:
        for a in bn_pack:
            args.append(a)
            in_specs.append(pl.BlockSpec((1, ci), lambda i: (0, 0)))
    for w in (Wls, Wfsb, Wfsg, *w_outs):
        args.append(w)
        in_specs.append(pl.BlockSpec(w.shape, lambda i: (0, 0)))
    for b in b_outs:
        if b is not None:
            args.append(b)
            in_specs.append(pl.BlockSpec(b.shape, lambda i: (0, 0)))

    co_skip = Wls.shape[0]
    out_shape = [jax.ShapeDtypeStruct((n, co_skip), jnp.float32)]
    out_specs = [pl.BlockSpec((bn, co_skip), lambda i: (i, 0))]
    for w in w_outs:
        out_shape.append(jax.ShapeDtypeStruct((n, w.shape[0]), jnp.float32))
        out_specs.append(pl.BlockSpec((bn, w.shape[0]), lambda i: (i, 0)))

    return pl.pallas_call(
        body, grid=grid, in_specs=in_specs, out_specs=out_specs,
        out_shape=out_shape)(*args)


def _combine(skip, agg, cnt):
    n, co = skip.shape
    bn = _row_block(n)
    grid = (n // bn,)

    def body(sr, ar, cr, yo, cso, cqo):
        c = jnp.maximum(cr[:, 0:1], 1.0)
        agg = jnp.concatenate([ar[0], ar[1], ar[2], ar[3]], axis=1)
        y = sr[...] + agg / c
        yo[...] = y

        @pl.when(pl.program_id(0) == 0)
        def _():
            cso[...] = jnp.zeros((1, co), jnp.float32)
            cqo[...] = jnp.zeros((1, co), jnp.float32)

        cso[...] += jnp.sum(y, axis=0, keepdims=True)
        cqo[...] += jnp.sum(y * y, axis=0, keepdims=True)

    return pl.pallas_call(
        body, grid=grid,
        in_specs=[pl.BlockSpec((bn, co), lambda i: (i, 0)),
                  pl.BlockSpec((4, bn, co // 4), lambda i: (0, i, 0)),
                  pl.BlockSpec((bn, _LANES), lambda i: (i, 0))],
        out_specs=[pl.BlockSpec((bn, co), lambda i: (i, 0)),
                   pl.BlockSpec((1, co), lambda i: (0, 0)),
                   pl.BlockSpec((1, co), lambda i: (0, 0))],
        out_shape=[jax.ShapeDtypeStruct((n, co), jnp.float32),
                   jax.ShapeDtypeStruct((1, co), jnp.float32),
                   jax.ShapeDtypeStruct((1, co), jnp.float32)])(skip, agg, cnt)


def _final_combine(skip3, g3, b3, p, cnt):
    n, d = skip3.shape
    bn = _row_block(n)
    grid = (n // bn,)

    def body(sr, gr, br, pr, cr, oo):
        s = pr[0] + pr[1]
        c = cr[...]
        oo[...] = (sr[...] + gr[...] * (s / jnp.maximum(c, 1.0))
                   + br[...] * jnp.minimum(c, 1.0))

    return pl.pallas_call(
        body, grid=grid,
        in_specs=[pl.BlockSpec((bn, d), lambda i: (i, 0)),
                  pl.BlockSpec((bn, d), lambda i: (i, 0)),
                  pl.BlockSpec((bn, d), lambda i: (i, 0)),
                  pl.BlockSpec((_CORES, bn, d), lambda i: (0, i, 0)),
                  pl.BlockSpec((bn, _LANES), lambda i: (i, 0))],
        out_specs=pl.BlockSpec((bn, d), lambda i: (i, 0)),
        out_shape=jax.ShapeDtypeStruct((n, d), jnp.float32))(
            skip3, g3, b3, p, cnt)


def _zero_stripe(buf, dst, row0, rpt):
    nfull = rpt // _CHUNK
    rem = rpt % _CHUNK
    for k in range(nfull):
        pltpu.sync_copy(buf, dst.at[pl.ds(row0 + k * _CHUNK, _CHUNK)])
    if rem:
        pltpu.sync_copy(buf.at[pl.ds(0, rem)],
                        dst.at[pl.ds(row0 + nfull * _CHUNK, rem)])


def _sc_count(dst_p, nacc):
    ep = dst_p.shape[0]
    per_tile = ep // _TILES
    nchunks = per_tile // _CHUNK
    rpt = nacc // _TILES

    mesh = plsc.VectorSubcoreMesh(core_axis_name="c", subcore_axis_name="s")

    def body(dstr, cnto, didx, ones, cacc, sem):
        cid = lax.axis_index("c")
        sid = lax.axis_index("s")
        row0 = sid * rpt

        @pl.when(cid == 0)
        def _():
            @pl.loop(0, _CHUNK)
            def _(e):
                ones[pl.ds(e, 1), pl.ds(0, _LANES)] = jnp.zeros(
                    (1, _LANES), jnp.float32)

            _zero_stripe(ones, cacc, row0, rpt)

            @pl.loop(0, _CHUNK)
            def _(e):
                ones[pl.ds(e, 1), pl.ds(0, _LANES)] = jnp.ones(
                    (1, _LANES), jnp.float32)

            plsc.subcore_barrier()

            @pl.loop(0, nchunks)
            def _(t):
                base = sid * per_tile + t * _CHUNK
                pltpu.sync_copy(dstr.at[pl.ds(base, _CHUNK)], didx)
                pltpu.sync_copy(ones, cacc.at[didx], add=True)

            plsc.subcore_barrier()
            pltpu.sync_copy(cacc.at[pl.ds(row0, rpt)],
                            cnto.at[pl.ds(row0, rpt)])

    scratch = [
        pltpu.VMEM((_CHUNK,), jnp.int32),
        pltpu.VMEM((_CHUNK, _LANES), jnp.float32),
        pltpu.VMEM_SHARED((nacc, _LANES), jnp.float32),
        pltpu.SemaphoreType.DMA,
    ]
    fn = pl.kernel(body,
                   out_type=jax.ShapeDtypeStruct((nacc, _LANES), jnp.float32),
                   mesh=mesh, scratch_types=scratch,
                   compiler_params=_SC_PARAMS)
    return fn(dst_p)


def _sc_edge(xws, gbs, src_p, dst_p, nacc):
    ep = src_p.shape[0]
    per_tile = ep // _TILES
    nchunks = per_tile // _CHUNK
    rpt = nacc // _TILES
    q = xws[0].shape[1]
    nj = q // _LANES

    mesh = plsc.VectorSubcoreMesh(core_axis_name="c", subcore_axis_name="s")

    def body(x0, x1, x2, x3, g0, g1, g2, g3, srcr, dstr, aggo,
             sidxa, sidxb, didxa, didxb, bufxa, bufxb, bufgba, bufgbb, acc,
             semxa, semga, semxb, semgb):
        cid = lax.axis_index("c")
        sid = lax.axis_index("s")
        row0 = sid * rpt
        ebase = sid * per_tile

        def compute(bx, bg):
            @pl.loop(0, _CHUNK)
            def _(e):
                for j in range(nj):
                    sl = (pl.ds(e, 1), pl.ds(j * _LANES, _LANES))
                    slb = (pl.ds(e, 1), pl.ds(q + j * _LANES, _LANES))
                    bx[sl] = jnp.maximum(bg[sl] * bx[sl] + bg[slb], 0.0)

        def run_quarter(xw, gb, plane):
            @pl.loop(0, _CHUNK)
            def _(e):
                for j in range(nj):
                    bufxa[pl.ds(e, 1), pl.ds(j * _LANES, _LANES)] = jnp.zeros(
                        (1, _LANES), jnp.float32)

            _zero_stripe(bufxa, acc, row0, rpt)
            plsc.subcore_barrier()

            @pl.loop(0, nchunks, step=2)
            def _(t):
                base = ebase + t * _CHUNK
                pltpu.sync_copy(srcr.at[pl.ds(base, _CHUNK)], sidxa)
                pltpu.sync_copy(srcr.at[pl.ds(base + _CHUNK, _CHUNK)], sidxb)
                pltpu.sync_copy(dstr.at[pl.ds(base, _CHUNK)], didxa)
                pltpu.sync_copy(dstr.at[pl.ds(base + _CHUNK, _CHUNK)], didxb)
                cxa = pltpu.async_copy(xw.at[sidxa], bufxa, semxa)
                cga = pltpu.async_copy(gb.at[didxa], bufgba, semga)
                cxb = pltpu.async_copy(xw.at[sidxb], bufxb, semxb)
                cgb = pltpu.async_copy(gb.at[didxb], bufgbb, semgb)
                cxa.wait()
                cga.wait()
                compute(bufxa, bufgba)
                pltpu.sync_copy(bufxa, acc.at[didxa], add=True)
                cxb.wait()
                cgb.wait()
                compute(bufxb, bufgbb)
                pltpu.sync_copy(bufxb, acc.at[didxb], add=True)

            plsc.subcore_barrier()
            pltpu.sync_copy(acc.at[pl.ds(row0, rpt)],
                            aggo.at[plane, pl.ds(row0, rpt)])

        @pl.when(cid == 0)
        def _():
            run_quarter(x0, g0, 0)
            run_quarter(x1, g1, 1)

        @pl.when(cid == 1)
        def _():
            run_quarter(x2, g2, 2)
            run_quarter(x3, g3, 3)

    scratch = [
        pltpu.VMEM((_CHUNK,), jnp.int32),
        pltpu.VMEM((_CHUNK,), jnp.int32),
        pltpu.VMEM((_CHUNK,), jnp.int32),
        pltpu.VMEM((_CHUNK,), jnp.int32),
        pltpu.VMEM((_CHUNK, q), jnp.float32),
        pltpu.VMEM((_CHUNK, q), jnp.float32),
        pltpu.VMEM((_CHUNK, 2 * q), jnp.float32),
        pltpu.VMEM((_CHUNK, 2 * q), jnp.float32),
        pltpu.VMEM_SHARED((nacc, q), jnp.float32),
        pltpu.SemaphoreType.DMA,
        pltpu.SemaphoreType.DMA,
        pltpu.SemaphoreType.DMA,
        pltpu.SemaphoreType.DMA,
    ]
    fn = pl.kernel(body,
                   out_type=jax.ShapeDtypeStruct((4, nacc, q), jnp.float32),
                   mesh=mesh, scratch_types=scratch,
                   compiler_params=_SC_PARAMS)
    return fn(*xws, *gbs, src_p, dst_p)


def _sc_final(xw3, src_p, dst_p, nacc):
    ep = src_p.shape[0]
    per_w = ep // (_TILES * _CORES)
    nchunks = per_w // _CHUNK
    rpt = nacc // _TILES
    d = xw3.shape[1]

    mesh = plsc.VectorSubcoreMesh(core_axis_name="c", subcore_axis_name="s")

    def body(xwr, srcr, dstr, po, sidx, didx, bufx, acc, sem):
        cid = lax.axis_index("c")
        sid = lax.axis_index("s")
        row0 = sid * rpt

        @pl.loop(0, _CHUNK)
        def _(e):
            bufx[pl.ds(e, 1), pl.ds(0, d)] = jnp.zeros((1, d), jnp.float32)

        _zero_stripe(bufx, acc, row0, rpt)
        plsc.subcore_barrier()

        wbase = (cid * _TILES + sid) * per_w

        @pl.loop(0, nchunks)
        def _(t):
            base = wbase + t * _CHUNK
            pltpu.sync_copy(srcr.at[pl.ds(base, _CHUNK)], sidx)
            pltpu.sync_copy(dstr.at[pl.ds(base, _CHUNK)], didx)
            pltpu.async_copy(xwr.at[sidx], bufx, sem).wait()
            pltpu.sync_copy(bufx, acc.at[didx], add=True)

        plsc.subcore_barrier()

        @pl.when(cid == 0)
        def _():
            pltpu.sync_copy(acc.at[pl.ds(row0, rpt)],
                            po.at[0, pl.ds(row0, rpt)])

        @pl.when(cid == 1)
        def _():
            pltpu.sync_copy(acc.at[pl.ds(row0, rpt)],
                            po.at[1, pl.ds(row0, rpt)])

    scratch = [
        pltpu.VMEM((_CHUNK,), jnp.int32),
        pltpu.VMEM((_CHUNK,), jnp.int32),
        pltpu.VMEM((_CHUNK, d), jnp.float32),
        pltpu.VMEM_SHARED((nacc, d), jnp.float32),
        pltpu.SemaphoreType.DMA,
    ]
    fn = pl.kernel(body, out_type=jax.ShapeDtypeStruct((_CORES, nacc, d),
                                                       jnp.float32),
                   mesh=mesh, scratch_types=scratch,
                   compiler_params=_SC_PARAMS)
    return fn(xw3, src_p, dst_p)


def _split_weights(W_lin, W_film, b_film, W_fs, split):
    co = W_lin.shape[0]
    Wfsb, Wfsg = W_fs[:co], W_fs[co:]
    Wfmb, Wfmg = W_film[:co], W_film[co:]
    bfb = b_film[:co].reshape(1, co)
    bfg = b_film[co:].reshape(1, co)
    if not split:
        return Wfsb, Wfsg, [Wfmb, Wfmg, W_lin], [bfb, bfg, None]
    h = co // 4
    w_outs, b_outs = [], []
    for k in range(4):
        w_outs.append(jnp.concatenate(
            [Wfmg[k * h:(k + 1) * h], Wfmb[k * h:(k + 1) * h]], axis=0))
        b_outs.append(jnp.concatenate(
            [bfg[:, k * h:(k + 1) * h], bfb[:, k * h:(k + 1) * h]], axis=1))
    for k in range(4):
        w_outs.append(W_lin[k * h:(k + 1) * h])
        b_outs.append(None)
    return Wfsb, Wfsg, w_outs, b_outs


def kernel(x, edge_index,
           W_lin0, W_film0, b_film0, W_ls0, W_fs0,
           W_lin1, W_film1, b_film1, W_ls1, W_fs1,
           W_lin2, W_film2, b_film2, W_ls2, W_fs2,
           W_lin3, W_film3, b_film3, W_ls3, W_fs3,
           bn_w0, bn_b0, bn_w1, bn_b1, bn_w2, bn_b2):
    n = x.shape[0]
    e = edge_index.shape[1]
    nacc = -(-(n + 1) // _TILES) * _TILES
    epad = _TILES * _CORES * _CHUNK
    ep = -(-e // epad) * epad
    src_p = jnp.concatenate(
        [edge_index[0], jnp.zeros((ep - e,), edge_index.dtype)])
    dst_p = jnp.concatenate(
        [edge_index[1], jnp.full((ep - e,), nacc - 1, edge_index.dtype)])

    cnt = _sc_count(dst_p, nacc)

    layer_w = [(W_lin0, W_film0, b_film0, W_ls0, W_fs0),
               (W_lin1, W_film1, b_film1, W_ls1, W_fs1),
               (W_lin2, W_film2, b_film2, W_ls2, W_fs2)]
    bn_params = [(bn_w0, bn_b0), (bn_w1, bn_b1), (bn_w2, bn_b2)]

    h = x
    bn_pack = None
    for i in range(3):
        W_lin, W_film, b_film, W_ls, W_fs = layer_w[i]
        Wfsb, Wfsg, w_outs, b_outs = _split_weights(W_lin, W_film, b_film,
                                                    W_fs, split=True)
        outs = _film_dense(
            h, bn_pack, W_ls, Wfsb, Wfsg, w_outs, b_outs, act=True)
        skip, gbq, xwq = outs[0], outs[1:5], outs[5:9]
        padrows = ((0, nacc - n), (0, 0))
        agg = _sc_edge(xwq, [jnp.pad(g, padrows) for g in gbq],
                       src_p, dst_p, nacc)
        y, cs, cq = _combine(skip, agg, cnt)
        bw = bn_params[i][0].reshape(1, y.shape[1])
        bb = bn_params[i][1].reshape(1, y.shape[1])
        bn_pack = (cs, cq, bw, bb)
        h = y

    Wfsb3, Wfsg3, w_outs3, b_outs3 = _split_weights(
        W_lin3, W_film3, b_film3, W_fs3, split=False)
    skip3, b3, g3, xw3 = _film_dense(
        h, bn_pack, W_ls3, Wfsb3, Wfsg3, w_outs3, b_outs3, act=False)
    p = _sc_final(xw3, src_p, dst_p, nacc)
    return _final_combine(skip3, g3, b3, p, cnt)


# speedup vs baseline: 1.1593x; 1.1593x over previous
import jax
import jax.numpy as jnp
from jax import lax
from jax.experimental import pallas as pl
from jax.experimental.pallas import tpu as pltpu
from jax.experimental.pallas import tpu_sc as plsc

_EPS = 1e-5
_CHUNK = 56
_IDXB = 8
_LANES = 16
_TILES = 16
_CORES = 2

_SC_PARAMS = pltpu.CompilerParams(use_tc_tiling_on_sc=False)


def _row_block(n):
    return 1000 if n % 1000 == 0 else n


def _film_dense(xin, bn_pack, Wls, Wfsb, Wfsg, w_outs, b_outs, act):
    n, ci = xin.shape
    bn = _row_block(n)
    grid = (n // bn,)
    nouts = len(w_outs)
    has_bias = [b is not None for b in b_outs]

    def body(*refs):
        refs = list(refs)
        if bn_pack is not None:
            yr, csr, cqr, bwr, bbr = refs[:5]
            del refs[:5]
            mu = csr[...] / n
            var = cqr[...] / n - mu * mu
            xb = (yr[...] - mu) * lax.rsqrt(var + _EPS) * bwr[...] + bbr[...]
        else:
            xb = refs.pop(0)[...]
        wls, wfsb, wfsg = refs[0], refs[1], refs[2]
        wrefs = refs[3:3 + nouts]
        brefs = refs[3 + nouts:3 + nouts + sum(has_bias)]
        outs = refs[3 + nouts + sum(has_bias):]

        def dot(wr):
            return lax.dot_general(xb, wr[...], (((1,), (1,)), ((), ())),
                                   preferred_element_type=jnp.float32)

        skip = dot(wfsg) * dot(wls) + dot(wfsb)
        if act:
            skip = jnp.maximum(skip, 0.0)
        outs[0][...] = skip
        bi = 0
        for k in range(nouts):
            r = dot(wrefs[k])
            if has_bias[k]:
                r = r + brefs[bi][...]
                bi += 1
            outs[1 + k][...] = r

    args = [xin]
    in_specs = [pl.BlockSpec((bn, ci), lambda i: (i, 0))]
    if bn_pack is not None:
        for a in bn_pack:
            args.append(a)
            in_specs.append(pl.BlockSpec((1, ci), lambda i: (0, 0)))
    for w in (Wls, Wfsb, Wfsg, *w_outs):
        args.append(w)
        in_specs.append(pl.BlockSpec(w.shape, lambda i: (0, 0)))
    for b in b_outs:
        if b is not None:
            args.append(b)
            in_specs.append(pl.BlockSpec(b.shape, lambda i: (0, 0)))

    co_skip = Wls.shape[0]
    out_shape = [jax.ShapeDtypeStruct((n, co_skip), jnp.float32)]
    out_specs = [pl.BlockSpec((bn, co_skip), lambda i: (i, 0))]
    for w in w_outs:
        out_shape.append(jax.ShapeDtypeStruct((n, w.shape[0]), jnp.float32))
        out_specs.append(pl.BlockSpec((bn, w.shape[0]), lambda i: (i, 0)))

    return pl.pallas_call(
        body, grid=grid, in_specs=in_specs, out_specs=out_specs,
        out_shape=out_shape)(*args)


def _combine(skip, agg, cnt):
    n, co = skip.shape
    bn = _row_block(n)
    grid = (n // bn,)

    def body(sr, ar, cr, yo, cso, cqo):
        c = jnp.maximum(cr[:, 0:1], 1.0)
        agg = jnp.concatenate([ar[0], ar[1]], axis=1)
        y = sr[...] + agg / c
        yo[...] = y

        @pl.when(pl.program_id(0) == 0)
        def _():
            cso[...] = jnp.zeros((1, co), jnp.float32)
            cqo[...] = jnp.zeros((1, co), jnp.float32)

        cso[...] += jnp.sum(y, axis=0, keepdims=True)
        cqo[...] += jnp.sum(y * y, axis=0, keepdims=True)

    return pl.pallas_call(
        body, grid=grid,
        in_specs=[pl.BlockSpec((bn, co), lambda i: (i, 0)),
                  pl.BlockSpec((_CORES, bn, co // 2), lambda i: (0, i, 0)),
                  pl.BlockSpec((bn, _LANES), lambda i: (i, 0))],
        out_specs=[pl.BlockSpec((bn, co), lambda i: (i, 0)),
                   pl.BlockSpec((1, co), lambda i: (0, 0)),
                   pl.BlockSpec((1, co), lambda i: (0, 0))],
        out_shape=[jax.ShapeDtypeStruct((n, co), jnp.float32),
                   jax.ShapeDtypeStruct((1, co), jnp.float32),
                   jax.ShapeDtypeStruct((1, co), jnp.float32)])(skip, agg, cnt)


def _final_combine(skip3, g3, b3, p, cnt):
    n, d = skip3.shape
    bn = _row_block(n)
    grid = (n // bn,)

    def body(sr, gr, br, pr, cr, oo):
        s = pr[0] + pr[1]
        c = cr[...]
        oo[...] = (sr[...] + gr[...] * (s / jnp.maximum(c, 1.0))
                   + br[...] * jnp.minimum(c, 1.0))

    return pl.pallas_call(
        body, grid=grid,
        in_specs=[pl.BlockSpec((bn, d), lambda i: (i, 0)),
                  pl.BlockSpec((bn, d), lambda i: (i, 0)),
                  pl.BlockSpec((bn, d), lambda i: (i, 0)),
                  pl.BlockSpec((_CORES, bn, d), lambda i: (0, i, 0)),
                  pl.BlockSpec((bn, _LANES), lambda i: (i, 0))],
        out_specs=pl.BlockSpec((bn, d), lambda i: (i, 0)),
        out_shape=jax.ShapeDtypeStruct((n, d), jnp.float32))(
            skip3, g3, b3, p, cnt)


def _zero_stripe(buf, dst, row0, rpt):
    nfull = rpt // _CHUNK
    rem = rpt % _CHUNK
    for k in range(nfull):
        pltpu.sync_copy(buf, dst.at[pl.ds(row0 + k * _CHUNK, _CHUNK)])
    if rem:
        pltpu.sync_copy(buf.at[pl.ds(0, rem)],
                        dst.at[pl.ds(row0 + nfull * _CHUNK, rem)])


def _sc_count(dst2, nacc):
    nrows = dst2.shape[0]
    rows_per_tile = nrows // _TILES
    nblocks = rows_per_tile // _IDXB
    rpt = nacc // _TILES

    mesh = plsc.VectorSubcoreMesh(core_axis_name="c", subcore_axis_name="s")

    def body(dstr, cnto, didx, ones, cacc, sem):
        cid = lax.axis_index("c")
        sid = lax.axis_index("s")
        row0 = sid * rpt

        @pl.when(cid == 0)
        def _():
            @pl.loop(0, _CHUNK)
            def _(e):
                ones[pl.ds(e, 1), pl.ds(0, _LANES)] = jnp.zeros(
                    (1, _LANES), jnp.float32)

            _zero_stripe(ones, cacc, row0, rpt)

            @pl.loop(0, _CHUNK)
            def _(e):
                ones[pl.ds(e, 1), pl.ds(0, _LANES)] = jnp.ones(
                    (1, _LANES), jnp.float32)

            plsc.subcore_barrier()

            @pl.loop(0, nblocks)
            def _(b):
                blk = sid * rows_per_tile + b * _IDXB
                pltpu.sync_copy(dstr.at[pl.ds(blk, _IDXB)], didx)

                @pl.loop(0, _IDXB)
                def _(j):
                    pltpu.sync_copy(ones, cacc.at[didx.at[j]], add=True)

            plsc.subcore_barrier()
            pltpu.sync_copy(cacc.at[pl.ds(row0, rpt)],
                            cnto.at[pl.ds(row0, rpt)])

    scratch = [
        pltpu.VMEM((_IDXB, _CHUNK), jnp.int32),
        pltpu.VMEM((_CHUNK, _LANES), jnp.float32),
        pltpu.VMEM_SHARED((nacc, _LANES), jnp.float32),
        pltpu.SemaphoreType.DMA,
    ]
    fn = pl.kernel(body,
                   out_type=jax.ShapeDtypeStruct((nacc, _LANES), jnp.float32),
                   mesh=mesh, scratch_types=scratch,
                   compiler_params=_SC_PARAMS)
    return fn(dst2)


def _sc_edge(xw_lo, xw_hi, gb_lo, gb_hi, src2, dst2, nacc):
    nrows = src2.shape[0]
    rows_per_tile = nrows // _TILES
    nblocks = rows_per_tile // _IDXB
    rpt = nacc // _TILES
    half = xw_lo.shape[1]
    nj = half // _LANES

    mesh = plsc.VectorSubcoreMesh(core_axis_name="c", subcore_axis_name="s")

    def body(xl, xh, gl, gh, srcr, dstr, aggo,
             sidx, didx, bufx, bufgb, acc, semx, semg):
        cid = lax.axis_index("c")
        sid = lax.axis_index("s")
        row0 = sid * rpt

        @pl.loop(0, _CHUNK)
        def _(e):
            for j in range(nj):
                bufx[pl.ds(e, 1), pl.ds(j * _LANES, _LANES)] = jnp.zeros(
                    (1, _LANES), jnp.float32)

        _zero_stripe(bufx, acc, row0, rpt)
        plsc.subcore_barrier()

        def run_half(xw, gb):
            rbase = sid * rows_per_tile

            @pl.loop(0, nblocks)
            def _(b):
                blk = rbase + b * _IDXB
                pltpu.sync_copy(srcr.at[pl.ds(blk, _IDXB)], sidx)
                pltpu.sync_copy(dstr.at[pl.ds(blk, _IDXB)], didx)

                @pl.loop(0, _IDXB)
                def _(j):
                    cx = pltpu.async_copy(xw.at[sidx.at[j]], bufx, semx)
                    cg = pltpu.async_copy(gb.at[didx.at[j]], bufgb, semg)
                    cx.wait()
                    cg.wait()

                    @pl.loop(0, _CHUNK)
                    def _(e):
                        for k in range(nj):
                            sl = (pl.ds(e, 1), pl.ds(k * _LANES, _LANES))
                            slb = (pl.ds(e, 1),
                                   pl.ds(half + k * _LANES, _LANES))
                            bufx[sl] = jnp.maximum(
                                bufgb[sl] * bufx[sl] + bufgb[slb], 0.0)

                    pltpu.sync_copy(bufx, acc.at[didx.at[j]], add=True)

        @pl.when(cid == 0)
        def _():
            run_half(xl, gl)

        @pl.when(cid == 1)
        def _():
            run_half(xh, gh)

        plsc.subcore_barrier()

        @pl.when(cid == 0)
        def _():
            pltpu.sync_copy(acc.at[pl.ds(row0, rpt)],
                            aggo.at[0, pl.ds(row0, rpt)])

        @pl.when(cid == 1)
        def _():
            pltpu.sync_copy(acc.at[pl.ds(row0, rpt)],
                            aggo.at[1, pl.ds(row0, rpt)])

    scratch = [
        pltpu.VMEM((_IDXB, _CHUNK), jnp.int32),
        pltpu.VMEM((_IDXB, _CHUNK), jnp.int32),
        pltpu.VMEM((_CHUNK, half), jnp.float32),
        pltpu.VMEM((_CHUNK, 2 * half), jnp.float32),
        pltpu.VMEM_SHARED((nacc, half), jnp.float32),
        pltpu.SemaphoreType.DMA,
        pltpu.SemaphoreType.DMA,
    ]
    fn = pl.kernel(body,
                   out_type=jax.ShapeDtypeStruct((_CORES, nacc, half),
                                                 jnp.float32),
                   mesh=mesh, scratch_types=scratch,
                   compiler_params=_SC_PARAMS)
    return fn(xw_lo, xw_hi, gb_lo, gb_hi, src2, dst2)


def _sc_final(xw3, src2, dst2, nacc):
    nrows = src2.shape[0]
    rows_per_w = nrows // (_TILES * _CORES)
    nb = 4 if rows_per_w % 4 == 0 else 1
    nblocks = rows_per_w // nb
    rpt = nacc // _TILES
    d = xw3.shape[1]

    mesh = plsc.VectorSubcoreMesh(core_axis_name="c", subcore_axis_name="s")

    def body(xwr, srcr, dstr, po, sidx, didx, bufx, acc, sem):
        cid = lax.axis_index("c")
        sid = lax.axis_index("s")
        row0 = sid * rpt

        @pl.loop(0, _CHUNK)
        def _(e):
            bufx[pl.ds(e, 1), pl.ds(0, d)] = jnp.zeros((1, d), jnp.float32)

        _zero_stripe(bufx, acc, row0, rpt)
        plsc.subcore_barrier()

        wbase = (cid * _TILES + sid) * rows_per_w

        @pl.loop(0, nblocks)
        def _(b):
            blk = wbase + b * nb
            pltpu.sync_copy(srcr.at[pl.ds(blk, nb)], sidx)
            pltpu.sync_copy(dstr.at[pl.ds(blk, nb)], didx)

            @pl.loop(0, nb)
            def _(j):
                pltpu.async_copy(xwr.at[sidx.at[j]], bufx, sem).wait()
                pltpu.sync_copy(bufx, acc.at[didx.at[j]], add=True)

        plsc.subcore_barrier()

        @pl.when(cid == 0)
        def _():
            pltpu.sync_copy(acc.at[pl.ds(row0, rpt)],
                            po.at[0, pl.ds(row0, rpt)])

        @pl.when(cid == 1)
        def _():
            pltpu.sync_copy(acc.at[pl.ds(row0, rpt)],
                            po.at[1, pl.ds(row0, rpt)])

    scratch = [
        pltpu.VMEM((nb, _CHUNK), jnp.int32),
        pltpu.VMEM((nb, _CHUNK), jnp.int32),
        pltpu.VMEM((_CHUNK, d), jnp.float32),
        pltpu.VMEM_SHARED((nacc, d), jnp.float32),
        pltpu.SemaphoreType.DMA,
    ]
    fn = pl.kernel(body, out_type=jax.ShapeDtypeStruct((_CORES, nacc, d),
                                                       jnp.float32),
                   mesh=mesh, scratch_types=scratch,
                   compiler_params=_SC_PARAMS)
    return fn(xw3, src2, dst2)


def _split_weights(W_lin, W_film, b_film, W_fs, split):
    co = W_lin.shape[0]
    Wfsb, Wfsg = W_fs[:co], W_fs[co:]
    Wfmb, Wfmg = W_film[:co], W_film[co:]
    bfb = b_film[:co].reshape(1, co)
    bfg = b_film[co:].reshape(1, co)
    if not split:
        return Wfsb, Wfsg, [Wfmb, Wfmg, W_lin], [bfb, bfg, None]
    h = co // 2
    w_outs, b_outs = [], []
    for k in range(2):
        w_outs.append(jnp.concatenate(
            [Wfmg[k * h:(k + 1) * h], Wfmb[k * h:(k + 1) * h]], axis=0))
        b_outs.append(jnp.concatenate(
            [bfg[:, k * h:(k + 1) * h], bfb[:, k * h:(k + 1) * h]], axis=1))
    for k in range(2):
        w_outs.append(W_lin[k * h:(k + 1) * h])
        b_outs.append(None)
    return Wfsb, Wfsg, w_outs, b_outs


def kernel(x, edge_index,
           W_lin0, W_film0, b_film0, W_ls0, W_fs0,
           W_lin1, W_film1, b_film1, W_ls1, W_fs1,
           W_lin2, W_film2, b_film2, W_ls2, W_fs2,
           W_lin3, W_film3, b_film3, W_ls3, W_fs3,
           bn_w0, bn_b0, bn_w1, bn_b1, bn_w2, bn_b2):
    n = x.shape[0]
    e = edge_index.shape[1]
    nacc = -(-(n + 1) // _TILES) * _TILES
    epad = _TILES * _IDXB * _CHUNK
    ep = -(-e // epad) * epad
    src_p = jnp.concatenate(
        [edge_index[0], jnp.zeros((ep - e,), edge_index.dtype)])
    dst_p = jnp.concatenate(
        [edge_index[1], jnp.full((ep - e,), nacc - 1, edge_index.dtype)])
    src2 = src_p.reshape(ep // _CHUNK, _CHUNK)
    dst2 = dst_p.reshape(ep // _CHUNK, _CHUNK)

    cnt = _sc_count(dst2, nacc)

    layer_w = [(W_lin0, W_film0, b_film0, W_ls0, W_fs0),
               (W_lin1, W_film1, b_film1, W_ls1, W_fs1),
               (W_lin2, W_film2, b_film2, W_ls2, W_fs2)]
    bn_params = [(bn_w0, bn_b0), (bn_w1, bn_b1), (bn_w2, bn_b2)]

    h = x
    bn_pack = None
    for i in range(3):
        W_lin, W_film, b_film, W_ls, W_fs = layer_w[i]
        Wfsb, Wfsg, w_outs, b_outs = _split_weights(W_lin, W_film, b_film,
                                                    W_fs, split=True)
        skip, gb_lo, gb_hi, xw_lo, xw_hi = _film_dense(
            h, bn_pack, W_ls, Wfsb, Wfsg, w_outs, b_outs, act=True)
        padrows = ((0, nacc - n), (0, 0))
        agg = _sc_edge(xw_lo, xw_hi,
                       jnp.pad(gb_lo, padrows), jnp.pad(gb_hi, padrows),
                       src2, dst2, nacc)
        y, cs, cq = _combine(skip, agg, cnt)
        bw = bn_params[i][0].reshape(1, y.shape[1])
        bb = bn_params[i][1].reshape(1, y.shape[1])
        bn_pack = (cs, cq, bw, bb)
        h = y

    Wfsb3, Wfsg3, w_outs3, b_outs3 = _split_weights(
        W_lin3, W_film3, b_film3, W_fs3, split=False)
    skip3, b3, g3, xw3 = _film_dense(
        h, bn_pack, W_ls3, Wfsb3, Wfsg3, w_outs3, b_outs3, act=False)
    p = _sc_final(xw3, src2, dst2, nacc)
    return _final_combine(skip3, g3, b3, p, cnt)


# speedup vs baseline: 1.6467x; 1.4205x over previous
import jax
import jax.numpy as jnp
from jax import lax
from jax.experimental import pallas as pl
from jax.experimental.pallas import tpu as pltpu
from jax.experimental.pallas import tpu_sc as plsc

_EPS = 1e-5
_CHUNK = 64
_LANES = 16
_TILES = 16
_CORES = 2

_SC_PARAMS = pltpu.CompilerParams(use_tc_tiling_on_sc=False)


def _row_block(n):
    return 1000 if n % 1000 == 0 else n


def _film_dense(xin, bn_pack, Wls, Wfsb, Wfsg, w_outs, b_outs, act):
    n, ci = xin.shape
    bn = _row_block(n)
    grid = (n // bn,)
    nouts = len(w_outs)
    has_bias = [b is not None for b in b_outs]

    def body(*refs):
        refs = list(refs)
        if bn_pack is not None:
            yr, csr, cqr, bwr, bbr = refs[:5]
            del refs[:5]
            mu = csr[...] / n
            var = cqr[...] / n - mu * mu
            xb = (yr[...] - mu) * lax.rsqrt(var + _EPS) * bwr[...] + bbr[...]
        else:
            xb = refs.pop(0)[...]
        wls, wfsb, wfsg = refs[0], refs[1], refs[2]
        wrefs = refs[3:3 + nouts]
        brefs = refs[3 + nouts:3 + nouts + sum(has_bias)]
        outs = refs[3 + nouts + sum(has_bias):]

        def dot(wr):
            return lax.dot_general(xb, wr[...], (((1,), (1,)), ((), ())),
                                   preferred_element_type=jnp.float32)

        skip = dot(wfsg) * dot(wls) + dot(wfsb)
        if act:
            skip = jnp.maximum(skip, 0.0)
        outs[0][...] = skip
        bi = 0
        for k in range(nouts):
            r = dot(wrefs[k])
            if has_bias[k]:
                r = r + brefs[bi][...]
                bi += 1
            outs[1 + k][...] = r

    args = [xin]
    in_specs = [pl.BlockSpec((bn, ci), lambda i: (i, 0))]
    if bn_pack is not None:
        for a in bn_pack:
            args.append(a)
            in_specs.append(pl.BlockSpec((1, ci), lambda i: (0, 0)))
    for w in (Wls, Wfsb, Wfsg, *w_outs):
        args.append(w)
        in_specs.append(pl.BlockSpec(w.shape, lambda i: (0, 0)))
    for b in b_outs:
        if b is not None:
            args.append(b)
            in_specs.append(pl.BlockSpec(b.shape, lambda i: (0, 0)))

    co_skip = Wls.shape[0]
    out_shape = [jax.ShapeDtypeStruct((n, co_skip), jnp.float32)]
    out_specs = [pl.BlockSpec((bn, co_skip), lambda i: (i, 0))]
    for w in w_outs:
        out_shape.append(jax.ShapeDtypeStruct((n, w.shape[0]), jnp.float32))
        out_specs.append(pl.BlockSpec((bn, w.shape[0]), lambda i: (i, 0)))

    return pl.pallas_call(
        body, grid=grid, in_specs=in_specs, out_specs=out_specs,
        out_shape=out_shape)(*args)


def _combine(skip, agg, cnt):
    n, co = skip.shape
    bn = _row_block(n)
    grid = (n // bn,)

    def body(sr, ar, cr, yo, cso, cqo):
        c = jnp.maximum(cr[:, 0:1], 1.0)
        agg = jnp.concatenate([ar[0], ar[1]], axis=1)
        y = sr[...] + agg / c
        yo[...] = y

        @pl.when(pl.program_id(0) == 0)
        def _():
            cso[...] = jnp.zeros((1, co), jnp.float32)
            cqo[...] = jnp.zeros((1, co), jnp.float32)

        cso[...] += jnp.sum(y, axis=0, keepdims=True)
        cqo[...] += jnp.sum(y * y, axis=0, keepdims=True)

    return pl.pallas_call(
        body, grid=grid,
        in_specs=[pl.BlockSpec((bn, co), lambda i: (i, 0)),
                  pl.BlockSpec((_CORES, bn, co // 2), lambda i: (0, i, 0)),
                  pl.BlockSpec((bn, _LANES), lambda i: (i, 0))],
        out_specs=[pl.BlockSpec((bn, co), lambda i: (i, 0)),
                   pl.BlockSpec((1, co), lambda i: (0, 0)),
                   pl.BlockSpec((1, co), lambda i: (0, 0))],
        out_shape=[jax.ShapeDtypeStruct((n, co), jnp.float32),
                   jax.ShapeDtypeStruct((1, co), jnp.float32),
                   jax.ShapeDtypeStruct((1, co), jnp.float32)])(skip, agg, cnt)


def _final_combine(skip3, g3, b3, p, cnt):
    n, d = skip3.shape
    bn = _row_block(n)
    grid = (n // bn,)

    def body(sr, gr, br, pr, cr, oo):
        s = pr[0] + pr[1]
        c = cr[...]
        oo[...] = (sr[...] + gr[...] * (s / jnp.maximum(c, 1.0))
                   + br[...] * jnp.minimum(c, 1.0))

    return pl.pallas_call(
        body, grid=grid,
        in_specs=[pl.BlockSpec((bn, d), lambda i: (i, 0)),
                  pl.BlockSpec((bn, d), lambda i: (i, 0)),
                  pl.BlockSpec((bn, d), lambda i: (i, 0)),
                  pl.BlockSpec((_CORES, bn, d), lambda i: (0, i, 0)),
                  pl.BlockSpec((bn, _LANES), lambda i: (i, 0))],
        out_specs=pl.BlockSpec((bn, d), lambda i: (i, 0)),
        out_shape=jax.ShapeDtypeStruct((n, d), jnp.float32))(
            skip3, g3, b3, p, cnt)


def _zero_stripe(buf, dst, row0, rpt):
    nfull = rpt // _CHUNK
    rem = rpt % _CHUNK
    for k in range(nfull):
        pltpu.sync_copy(buf, dst.at[pl.ds(row0 + k * _CHUNK, _CHUNK)])
    if rem:
        pltpu.sync_copy(buf.at[pl.ds(0, rem)],
                        dst.at[pl.ds(row0 + nfull * _CHUNK, rem)])


def _sc_count(dst_p, nacc):
    ep = dst_p.shape[0]
    per_tile = ep // _TILES
    nchunks = per_tile // _CHUNK
    rpt = nacc // _TILES

    mesh = plsc.VectorSubcoreMesh(core_axis_name="c", subcore_axis_name="s")

    def body(dstr, cnto, didx, ones, cacc, sem):
        cid = lax.axis_index("c")
        sid = lax.axis_index("s")
        row0 = sid * rpt

        @pl.when(cid == 0)
        def _():
            @pl.loop(0, _CHUNK)
            def _(e):
                ones[pl.ds(e, 1), pl.ds(0, _LANES)] = jnp.zeros(
                    (1, _LANES), jnp.float32)

            _zero_stripe(ones, cacc, row0, rpt)

            @pl.loop(0, _CHUNK)
            def _(e):
                ones[pl.ds(e, 1), pl.ds(0, _LANES)] = jnp.ones(
                    (1, _LANES), jnp.float32)

            plsc.subcore_barrier()

            @pl.loop(0, nchunks)
            def _(t):
                base = sid * per_tile + t * _CHUNK
                pltpu.sync_copy(dstr.at[pl.ds(base, _CHUNK)], didx)
                pltpu.sync_copy(ones, cacc.at[didx], add=True)

            plsc.subcore_barrier()
            pltpu.sync_copy(cacc.at[pl.ds(row0, rpt)],
                            cnto.at[pl.ds(row0, rpt)])

    scratch = [
        pltpu.VMEM((_CHUNK,), jnp.int32),
        pltpu.VMEM((_CHUNK, _LANES), jnp.float32),
        pltpu.VMEM_SHARED((nacc, _LANES), jnp.float32),
        pltpu.SemaphoreType.DMA,
    ]
    fn = pl.kernel(body,
                   out_type=jax.ShapeDtypeStruct((nacc, _LANES), jnp.float32),
                   mesh=mesh, scratch_types=scratch,
                   compiler_params=_SC_PARAMS)
    return fn(dst_p)


def _sc_edge(xw_lo, xw_hi, g_lo, g_hi, b_lo, b_hi, src_p, dst_p, nacc):
    ep = src_p.shape[0]
    per_tile = ep // _TILES
    nchunks = per_tile // _CHUNK
    rpt = nacc // _TILES
    half = xw_lo.shape[1]
    nj = half // _LANES

    mesh = plsc.VectorSubcoreMesh(core_axis_name="c", subcore_axis_name="s")

    def body(xl, xh, gl, gh, bl, bh, srcr, dstr, aggo,
             sidx, didx, bufx, bufg, bufb, acc, sem):
        cid = lax.axis_index("c")
        sid = lax.axis_index("s")
        row0 = sid * rpt

        @pl.loop(0, _CHUNK)
        def _(e):
            for j in range(nj):
                bufx[pl.ds(e, 1), pl.ds(j * _LANES, _LANES)] = jnp.zeros(
                    (1, _LANES), jnp.float32)

        _zero_stripe(bufx, acc, row0, rpt)
        plsc.subcore_barrier()

        def run_half(xw, g, b):
            ebase = sid * per_tile

            @pl.loop(0, nchunks)
            def _(t):
                base = ebase + t * _CHUNK
                pltpu.sync_copy(srcr.at[pl.ds(base, _CHUNK)], sidx)
                pltpu.sync_copy(dstr.at[pl.ds(base, _CHUNK)], didx)
                cx = pltpu.async_copy(xw.at[sidx], bufx, sem)
                cg = pltpu.async_copy(g.at[didx], bufg, sem)
                cb = pltpu.async_copy(b.at[didx], bufb, sem)
                cx.wait()
                cg.wait()
                cb.wait()

                @pl.loop(0, _CHUNK)
                def _(e):
                    for j in range(nj):
                        sl = (pl.ds(e, 1), pl.ds(j * _LANES, _LANES))
                        bufx[sl] = jnp.maximum(
                            bufg[sl] * bufx[sl] + bufb[sl], 0.0)

                pltpu.sync_copy(bufx, acc.at[didx], add=True)

        @pl.when(cid == 0)
        def _():
            run_half(xl, gl, bl)

        @pl.when(cid == 1)
        def _():
            run_half(xh, gh, bh)

        plsc.subcore_barrier()

        @pl.when(cid == 0)
        def _():
            pltpu.sync_copy(acc.at[pl.ds(row0, rpt)],
                            aggo.at[0, pl.ds(row0, rpt)])

        @pl.when(cid == 1)
        def _():
            pltpu.sync_copy(acc.at[pl.ds(row0, rpt)],
                            aggo.at[1, pl.ds(row0, rpt)])

    scratch = [
        pltpu.VMEM((_CHUNK,), jnp.int32),
        pltpu.VMEM((_CHUNK,), jnp.int32),
        pltpu.VMEM((_CHUNK, half), jnp.float32),
        pltpu.VMEM((_CHUNK, half), jnp.float32),
        pltpu.VMEM((_CHUNK, half), jnp.float32),
        pltpu.VMEM_SHARED((nacc, half), jnp.float32),
        pltpu.SemaphoreType.DMA,
    ]
    fn = pl.kernel(body,
                   out_type=jax.ShapeDtypeStruct((_CORES, nacc, half),
                                                 jnp.float32),
                   mesh=mesh, scratch_types=scratch,
                   compiler_params=_SC_PARAMS)
    return fn(xw_lo, xw_hi, g_lo, g_hi, b_lo, b_hi, src_p, dst_p)


def _sc_final(xw3, src_p, dst_p, nacc):
    ep = src_p.shape[0]
    per_w = ep // (_TILES * _CORES)
    nchunks = per_w // _CHUNK
    rpt = nacc // _TILES
    d = xw3.shape[1]

    mesh = plsc.VectorSubcoreMesh(core_axis_name="c", subcore_axis_name="s")

    def body(xwr, srcr, dstr, po, sidx, didx, bufx, acc, sem):
        cid = lax.axis_index("c")
        sid = lax.axis_index("s")
        row0 = sid * rpt

        @pl.loop(0, _CHUNK)
        def _(e):
            bufx[pl.ds(e, 1), pl.ds(0, d)] = jnp.zeros((1, d), jnp.float32)

        _zero_stripe(bufx, acc, row0, rpt)
        plsc.subcore_barrier()

        wbase = (cid * _TILES + sid) * per_w

        @pl.loop(0, nchunks)
        def _(t):
            base = wbase + t * _CHUNK
            pltpu.sync_copy(srcr.at[pl.ds(base, _CHUNK)], sidx)
            pltpu.sync_copy(dstr.at[pl.ds(base, _CHUNK)], didx)
            pltpu.async_copy(xwr.at[sidx], bufx, sem).wait()
            pltpu.sync_copy(bufx, acc.at[didx], add=True)

        plsc.subcore_barrier()

        @pl.when(cid == 0)
        def _():
            pltpu.sync_copy(acc.at[pl.ds(row0, rpt)],
                            po.at[0, pl.ds(row0, rpt)])

        @pl.when(cid == 1)
        def _():
            pltpu.sync_copy(acc.at[pl.ds(row0, rpt)],
                            po.at[1, pl.ds(row0, rpt)])

    scratch = [
        pltpu.VMEM((_CHUNK,), jnp.int32),
        pltpu.VMEM((_CHUNK,), jnp.int32),
        pltpu.VMEM((_CHUNK, d), jnp.float32),
        pltpu.VMEM_SHARED((nacc, d), jnp.float32),
        pltpu.SemaphoreType.DMA,
    ]
    fn = pl.kernel(body, out_type=jax.ShapeDtypeStruct((_CORES, nacc, d),
                                                       jnp.float32),
                   mesh=mesh, scratch_types=scratch,
                   compiler_params=_SC_PARAMS)
    return fn(xw3, src_p, dst_p)


def _split_weights(W_lin, W_film, b_film, W_fs, split):
    co = W_lin.shape[0]
    Wfsb, Wfsg = W_fs[:co], W_fs[co:]
    Wfmb, Wfmg = W_film[:co], W_film[co:]
    bfb = b_film[:co].reshape(1, co)
    bfg = b_film[co:].reshape(1, co)
    if not split:
        return Wfsb, Wfsg, [Wfmb, Wfmg, W_lin], [bfb, bfg, None]
    h = co // 2
    w_outs = [Wfmb[:h], Wfmb[h:], Wfmg[:h], Wfmg[h:], W_lin[:h], W_lin[h:]]
    b_outs = [bfb[:, :h], bfb[:, h:], bfg[:, :h], bfg[:, h:], None, None]
    return Wfsb, Wfsg, w_outs, b_outs


def kernel(x, edge_index,
           W_lin0, W_film0, b_film0, W_ls0, W_fs0,
           W_lin1, W_film1, b_film1, W_ls1, W_fs1,
           W_lin2, W_film2, b_film2, W_ls2, W_fs2,
           W_lin3, W_film3, b_film3, W_ls3, W_fs3,
           bn_w0, bn_b0, bn_w1, bn_b1, bn_w2, bn_b2):
    n = x.shape[0]
    e = edge_index.shape[1]
    nacc = -(-(n + 1) // _TILES) * _TILES
    epad = _TILES * _CORES * _CHUNK
    ep = -(-e // epad) * epad
    src_p = jnp.concatenate(
        [edge_index[0], jnp.zeros((ep - e,), edge_index.dtype)])
    dst_p = jnp.concatenate(
        [edge_index[1], jnp.full((ep - e,), nacc - 1, edge_index.dtype)])

    cnt = _sc_count(dst_p, nacc)

    layer_w = [(W_lin0, W_film0, b_film0, W_ls0, W_fs0),
               (W_lin1, W_film1, b_film1, W_ls1, W_fs1),
               (W_lin2, W_film2, b_film2, W_ls2, W_fs2)]
    bn_params = [(bn_w0, bn_b0), (bn_w1, bn_b1), (bn_w2, bn_b2)]

    h = x
    bn_pack = None
    for i in range(3):
        W_lin, W_film, b_film, W_ls, W_fs = layer_w[i]
        Wfsb, Wfsg, w_outs, b_outs = _split_weights(W_lin, W_film, b_film,
                                                    W_fs, split=True)
        skip, bl, bh, gl, gh, xl, xh = _film_dense(
            h, bn_pack, W_ls, Wfsb, Wfsg, w_outs, b_outs, act=True)
        padrows = ((0, nacc - n), (0, 0))
        agg = _sc_edge(xl, xh,
                       jnp.pad(gl, padrows), jnp.pad(gh, padrows),
                       jnp.pad(bl, padrows), jnp.pad(bh, padrows),
                       src_p, dst_p, nacc)
        y, cs, cq = _combine(skip, agg, cnt)
        bw = bn_params[i][0].reshape(1, y.shape[1])
        bb = bn_params[i][1].reshape(1, y.shape[1])
        bn_pack = (cs, cq, bw, bb)
        h = y

    Wfsb3, Wfsg3, w_outs3, b_outs3 = _split_weights(
        W_lin3, W_film3, b_film3, W_fs3, split=False)
    skip3, b3, g3, xw3 = _film_dense(
        h, bn_pack, W_ls3, Wfsb3, Wfsg3, w_outs3, b_outs3, act=False)
    p = _sc_final(xw3, src_p, dst_p, nacc)
    return _final_combine(skip3, g3, b3, p, cnt)


# speedup vs baseline: 1.7338x; 1.0529x over previous
import jax
import jax.numpy as jnp
from jax import lax
from jax.experimental import pallas as pl
from jax.experimental.pallas import tpu as pltpu
from jax.experimental.pallas import tpu_sc as plsc

_EPS = 1e-5
_CHUNK = 56
_IDXB = 8
_LANES = 16
_TILES = 16
_CORES = 2

_SC_PARAMS = pltpu.CompilerParams(use_tc_tiling_on_sc=False)


def _row_block(n):
    return 1000 if n % 1000 == 0 else n


def _film_dense(xin, bn_pack, Wls, Wfsb, Wfsg, w_outs, b_outs, act):
    n, ci = xin.shape
    bn = _row_block(n)
    grid = (n // bn,)
    nouts = len(w_outs)
    has_bias = [b is not None for b in b_outs]

    def body(*refs):
        refs = list(refs)
        if bn_pack is not None:
            yr, csr, cqr, bwr, bbr = refs[:5]
            del refs[:5]
            mu = csr[...] / n
            var = cqr[...] / n - mu * mu
            xb = (yr[...] - mu) * lax.rsqrt(var + _EPS) * bwr[...] + bbr[...]
        else:
            xb = refs.pop(0)[...]
        wls, wfsb, wfsg = refs[0], refs[1], refs[2]
        wrefs = refs[3:3 + nouts]
        brefs = refs[3 + nouts:3 + nouts + sum(has_bias)]
        outs = refs[3 + nouts + sum(has_bias):]

        def dot(wr):
            return lax.dot_general(xb, wr[...], (((1,), (1,)), ((), ())),
                                   preferred_element_type=jnp.float32)

        skip = dot(wfsg) * dot(wls) + dot(wfsb)
        if act:
            skip = jnp.maximum(skip, 0.0)
        outs[0][...] = skip
        bi = 0
        for k in range(nouts):
            r = dot(wrefs[k])
            if has_bias[k]:
                r = r + brefs[bi][...]
                bi += 1
            outs[1 + k][...] = r

    args = [xin]
    in_specs = [pl.BlockSpec((bn, ci), lambda i: (i, 0))]
    if bn_pack is not None:
        for a in bn_pack:
            args.append(a)
            in_specs.append(pl.BlockSpec((1, ci), lambda i: (0, 0)))
    for w in (Wls, Wfsb, Wfsg, *w_outs):
        args.append(w)
        in_specs.append(pl.BlockSpec(w.shape, lambda i: (0, 0)))
    for b in b_outs:
        if b is not None:
            args.append(b)
            in_specs.append(pl.BlockSpec(b.shape, lambda i: (0, 0)))

    co_skip = Wls.shape[0]
    out_shape = [jax.ShapeDtypeStruct((n, co_skip), jnp.float32)]
    out_specs = [pl.BlockSpec((bn, co_skip), lambda i: (i, 0))]
    for w in w_outs:
        out_shape.append(jax.ShapeDtypeStruct((n, w.shape[0]), jnp.float32))
        out_specs.append(pl.BlockSpec((bn, w.shape[0]), lambda i: (i, 0)))

    return pl.pallas_call(
        body, grid=grid, in_specs=in_specs, out_specs=out_specs,
        out_shape=out_shape)(*args)


def _combine(skip, agg, cnt):
    n, co = skip.shape
    bn = _row_block(n)
    grid = (n // bn,)

    def body(sr, ar, cr, yo, cso, cqo):
        c = jnp.maximum(cr[:, 0:1], 1.0)
        agg = jnp.concatenate([ar[0], ar[1]], axis=1)
        y = sr[...] + agg / c
        yo[...] = y

        @pl.when(pl.program_id(0) == 0)
        def _():
            cso[...] = jnp.zeros((1, co), jnp.float32)
            cqo[...] = jnp.zeros((1, co), jnp.float32)

        cso[...] += jnp.sum(y, axis=0, keepdims=True)
        cqo[...] += jnp.sum(y * y, axis=0, keepdims=True)

    return pl.pallas_call(
        body, grid=grid,
        in_specs=[pl.BlockSpec((bn, co), lambda i: (i, 0)),
                  pl.BlockSpec((_CORES, bn, co // 2), lambda i: (0, i, 0)),
                  pl.BlockSpec((bn, _LANES), lambda i: (i, 0))],
        out_specs=[pl.BlockSpec((bn, co), lambda i: (i, 0)),
                   pl.BlockSpec((1, co), lambda i: (0, 0)),
                   pl.BlockSpec((1, co), lambda i: (0, 0))],
        out_shape=[jax.ShapeDtypeStruct((n, co), jnp.float32),
                   jax.ShapeDtypeStruct((1, co), jnp.float32),
                   jax.ShapeDtypeStruct((1, co), jnp.float32)])(skip, agg, cnt)


def _final_combine(skip3, g3, b3, p, cnt):
    n, d = skip3.shape
    bn = _row_block(n)
    grid = (n // bn,)

    def body(sr, gr, br, pr, cr, oo):
        s = pr[0] + pr[1]
        c = cr[...]
        oo[...] = (sr[...] + gr[...] * (s / jnp.maximum(c, 1.0))
                   + br[...] * jnp.minimum(c, 1.0))

    return pl.pallas_call(
        body, grid=grid,
        in_specs=[pl.BlockSpec((bn, d), lambda i: (i, 0)),
                  pl.BlockSpec((bn, d), lambda i: (i, 0)),
                  pl.BlockSpec((bn, d), lambda i: (i, 0)),
                  pl.BlockSpec((_CORES, bn, d), lambda i: (0, i, 0)),
                  pl.BlockSpec((bn, _LANES), lambda i: (i, 0))],
        out_specs=pl.BlockSpec((bn, d), lambda i: (i, 0)),
        out_shape=jax.ShapeDtypeStruct((n, d), jnp.float32))(
            skip3, g3, b3, p, cnt)


def _zero_stripe(buf, dst, row0, rpt):
    nfull = rpt // _CHUNK
    rem = rpt % _CHUNK
    for k in range(nfull):
        pltpu.sync_copy(buf, dst.at[pl.ds(row0 + k * _CHUNK, _CHUNK)])
    if rem:
        pltpu.sync_copy(buf.at[pl.ds(0, rem)],
                        dst.at[pl.ds(row0 + nfull * _CHUNK, rem)])


def _sc_count(dst_p, nacc):
    ep = dst_p.shape[0]
    per_tile = ep // _TILES
    nchunks = per_tile // _CHUNK
    rpt = nacc // _TILES

    mesh = plsc.VectorSubcoreMesh(core_axis_name="c", subcore_axis_name="s")

    def body(dstr, cnto, didx, ones, cacc, sem):
        cid = lax.axis_index("c")
        sid = lax.axis_index("s")
        row0 = sid * rpt

        @pl.when(cid == 0)
        def _():
            @pl.loop(0, _CHUNK)
            def _(e):
                ones[pl.ds(e, 1), pl.ds(0, _LANES)] = jnp.zeros(
                    (1, _LANES), jnp.float32)

            _zero_stripe(ones, cacc, row0, rpt)

            @pl.loop(0, _CHUNK)
            def _(e):
                ones[pl.ds(e, 1), pl.ds(0, _LANES)] = jnp.ones(
                    (1, _LANES), jnp.float32)

            plsc.subcore_barrier()

            @pl.loop(0, nchunks)
            def _(t):
                base = sid * per_tile + t * _CHUNK
                pltpu.sync_copy(dstr.at[pl.ds(base, _CHUNK)], didx)
                pltpu.sync_copy(ones, cacc.at[didx], add=True)

            plsc.subcore_barrier()
            pltpu.sync_copy(cacc.at[pl.ds(row0, rpt)],
                            cnto.at[pl.ds(row0, rpt)])

    scratch = [
        pltpu.VMEM((_CHUNK,), jnp.int32),
        pltpu.VMEM((_CHUNK, _LANES), jnp.float32),
        pltpu.VMEM_SHARED((nacc, _LANES), jnp.float32),
        pltpu.SemaphoreType.DMA,
    ]
    fn = pl.kernel(body,
                   out_type=jax.ShapeDtypeStruct((nacc, _LANES), jnp.float32),
                   mesh=mesh, scratch_types=scratch,
                   compiler_params=_SC_PARAMS)
    return fn(dst_p)


def _sc_edge(xw_lo, xw_hi, g_lo, g_hi, b_lo, b_hi, src_p, dst_p, nacc):
    ep = src_p.shape[0]
    src2 = src_p.reshape(ep // _CHUNK, _CHUNK)
    dst2 = dst_p.reshape(ep // _CHUNK, _CHUNK)
    nrows = ep // _CHUNK
    rows_per_tile = nrows // _TILES
    nblocks = rows_per_tile // _IDXB
    rpt = nacc // _TILES
    half = xw_lo.shape[1]
    nj = half // _LANES

    mesh = plsc.VectorSubcoreMesh(core_axis_name="c", subcore_axis_name="s")

    def body(xl, xh, gl, gh, bl, bh, srcr, dstr, aggo,
             sidx, didx, bufx, bufg, bufb, acc, sem):
        cid = lax.axis_index("c")
        sid = lax.axis_index("s")
        row0 = sid * rpt

        @pl.loop(0, _CHUNK)
        def _(e):
            for j in range(nj):
                bufx[pl.ds(e, 1), pl.ds(j * _LANES, _LANES)] = jnp.zeros(
                    (1, _LANES), jnp.float32)

        _zero_stripe(bufx, acc, row0, rpt)
        plsc.subcore_barrier()

        def run_half(xw, g, b):
            rbase = sid * rows_per_tile

            @pl.loop(0, nblocks)
            def _(t):
                blk = rbase + t * _IDXB
                pltpu.sync_copy(srcr.at[pl.ds(blk, _IDXB)], sidx)
                pltpu.sync_copy(dstr.at[pl.ds(blk, _IDXB)], didx)

                @pl.loop(0, _IDXB)
                def _(j):
                    cx = pltpu.async_copy(xw.at[sidx.at[j]], bufx, sem)
                    cg = pltpu.async_copy(g.at[didx.at[j]], bufg, sem)
                    cb = pltpu.async_copy(b.at[didx.at[j]], bufb, sem)
                    cx.wait()
                    cg.wait()
                    cb.wait()

                    @pl.loop(0, _CHUNK)
                    def _(e):
                        for k in range(nj):
                            sl = (pl.ds(e, 1), pl.ds(k * _LANES, _LANES))
                            bufx[sl] = jnp.maximum(
                                bufg[sl] * bufx[sl] + bufb[sl], 0.0)

                    pltpu.sync_copy(bufx, acc.at[didx.at[j]], add=True)

        @pl.when(cid == 0)
        def _():
            run_half(xl, gl, bl)

        @pl.when(cid == 1)
        def _():
            run_half(xh, gh, bh)

        plsc.subcore_barrier()

        @pl.when(cid == 0)
        def _():
            pltpu.sync_copy(acc.at[pl.ds(row0, rpt)],
                            aggo.at[0, pl.ds(row0, rpt)])

        @pl.when(cid == 1)
        def _():
            pltpu.sync_copy(acc.at[pl.ds(row0, rpt)],
                            aggo.at[1, pl.ds(row0, rpt)])

    scratch = [
        pltpu.VMEM((_IDXB, _CHUNK), jnp.int32),
        pltpu.VMEM((_IDXB, _CHUNK), jnp.int32),
        pltpu.VMEM((_CHUNK, half), jnp.float32),
        pltpu.VMEM((_CHUNK, half), jnp.float32),
        pltpu.VMEM((_CHUNK, half), jnp.float32),
        pltpu.VMEM_SHARED((nacc, half), jnp.float32),
        pltpu.SemaphoreType.DMA,
    ]
    fn = pl.kernel(body,
                   out_type=jax.ShapeDtypeStruct((_CORES, nacc, half),
                                                 jnp.float32),
                   mesh=mesh, scratch_types=scratch,
                   compiler_params=_SC_PARAMS)
    return fn(xw_lo, xw_hi, g_lo, g_hi, b_lo, b_hi, src2, dst2)


def _sc_final(xw3, src_p, dst_p, nacc):
    ep = src_p.shape[0]
    per_w = ep // (_TILES * _CORES)
    nchunks = per_w // _CHUNK
    rpt = nacc // _TILES
    d = xw3.shape[1]

    mesh = plsc.VectorSubcoreMesh(core_axis_name="c", subcore_axis_name="s")

    def body(xwr, srcr, dstr, po, sidx, didx, bufx, acc, sem):
        cid = lax.axis_index("c")
        sid = lax.axis_index("s")
        row0 = sid * rpt

        @pl.loop(0, _CHUNK)
        def _(e):
            bufx[pl.ds(e, 1), pl.ds(0, d)] = jnp.zeros((1, d), jnp.float32)

        _zero_stripe(bufx, acc, row0, rpt)
        plsc.subcore_barrier()

        wbase = (cid * _TILES + sid) * per_w

        @pl.loop(0, nchunks)
        def _(t):
            base = wbase + t * _CHUNK
            pltpu.sync_copy(srcr.at[pl.ds(base, _CHUNK)], sidx)
            pltpu.sync_copy(dstr.at[pl.ds(base, _CHUNK)], didx)
            pltpu.async_copy(xwr.at[sidx], bufx, sem).wait()
            pltpu.sync_copy(bufx, acc.at[didx], add=True)

        plsc.subcore_barrier()

        @pl.when(cid == 0)
        def _():
            pltpu.sync_copy(acc.at[pl.ds(row0, rpt)],
                            po.at[0, pl.ds(row0, rpt)])

        @pl.when(cid == 1)
        def _():
            pltpu.sync_copy(acc.at[pl.ds(row0, rpt)],
                            po.at[1, pl.ds(row0, rpt)])

    scratch = [
        pltpu.VMEM((_CHUNK,), jnp.int32),
        pltpu.VMEM((_CHUNK,), jnp.int32),
        pltpu.VMEM((_CHUNK, d), jnp.float32),
        pltpu.VMEM_SHARED((nacc, d), jnp.float32),
        pltpu.SemaphoreType.DMA,
    ]
    fn = pl.kernel(body, out_type=jax.ShapeDtypeStruct((_CORES, nacc, d),
                                                       jnp.float32),
                   mesh=mesh, scratch_types=scratch,
                   compiler_params=_SC_PARAMS)
    return fn(xw3, src_p, dst_p)


def _split_weights(W_lin, W_film, b_film, W_fs, split):
    co = W_lin.shape[0]
    Wfsb, Wfsg = W_fs[:co], W_fs[co:]
    Wfmb, Wfmg = W_film[:co], W_film[co:]
    bfb = b_film[:co].reshape(1, co)
    bfg = b_film[co:].reshape(1, co)
    if not split:
        return Wfsb, Wfsg, [Wfmb, Wfmg, W_lin], [bfb, bfg, None]
    h = co // 2
    w_outs = [Wfmb[:h], Wfmb[h:], Wfmg[:h], Wfmg[h:], W_lin[:h], W_lin[h:]]
    b_outs = [bfb[:, :h], bfb[:, h:], bfg[:, :h], bfg[:, h:], None, None]
    return Wfsb, Wfsg, w_outs, b_outs


def kernel(x, edge_index,
           W_lin0, W_film0, b_film0, W_ls0, W_fs0,
           W_lin1, W_film1, b_film1, W_ls1, W_fs1,
           W_lin2, W_film2, b_film2, W_ls2, W_fs2,
           W_lin3, W_film3, b_film3, W_ls3, W_fs3,
           bn_w0, bn_b0, bn_w1, bn_b1, bn_w2, bn_b2):
    n = x.shape[0]
    e = edge_index.shape[1]
    nacc = -(-(n + 1) // _TILES) * _TILES
    epad = _TILES * _IDXB * _CHUNK
    ep = -(-e // epad) * epad
    src_p = jnp.concatenate(
        [edge_index[0], jnp.zeros((ep - e,), edge_index.dtype)])
    dst_p = jnp.concatenate(
        [edge_index[1], jnp.full((ep - e,), nacc - 1, edge_index.dtype)])

    cnt = _sc_count(dst_p, nacc)

    layer_w = [(W_lin0, W_film0, b_film0, W_ls0, W_fs0),
               (W_lin1, W_film1, b_film1, W_ls1, W_fs1),
               (W_lin2, W_film2, b_film2, W_ls2, W_fs2)]
    bn_params = [(bn_w0, bn_b0), (bn_w1, bn_b1), (bn_w2, bn_b2)]

    h = x
    bn_pack = None
    for i in range(3):
        W_lin, W_film, b_film, W_ls, W_fs = layer_w[i]
        Wfsb, Wfsg, w_outs, b_outs = _split_weights(W_lin, W_film, b_film,
                                                    W_fs, split=True)
        skip, bl, bh, gl, gh, xl, xh = _film_dense(
            h, bn_pack, W_ls, Wfsb, Wfsg, w_outs, b_outs, act=True)
        padrows = ((0, nacc - n), (0, 0))
        agg = _sc_edge(xl, xh,
                       jnp.pad(gl, padrows), jnp.pad(gh, padrows),
                       jnp.pad(bl, padrows), jnp.pad(bh, padrows),
                       src_p, dst_p, nacc)
        y, cs, cq = _combine(skip, agg, cnt)
        bw = bn_params[i][0].reshape(1, y.shape[1])
        bb = bn_params[i][1].reshape(1, y.shape[1])
        bn_pack = (cs, cq, bw, bb)
        h = y

    Wfsb3, Wfsg3, w_outs3, b_outs3 = _split_weights(
        W_lin3, W_film3, b_film3, W_fs3, split=False)
    skip3, b3, g3, xw3 = _film_dense(
        h, bn_pack, W_ls3, Wfsb3, Wfsg3, w_outs3, b_outs3, act=False)
    p = _sc_final(xw3, src_p, dst_p, nacc)
    return _final_combine(skip3, g3, b3, p, cnt)


# speedup vs baseline: 1.8195x; 1.0494x over previous
import jax
import jax.numpy as jnp
from jax import lax
from jax.experimental import pallas as pl
from jax.experimental.pallas import tpu as pltpu
from jax.experimental.pallas import tpu_sc as plsc

_EPS = 1e-5
_CHUNK = 56
_IDXB = 8
_LANES = 16
_TILES = 16
_CORES = 2

_SC_PARAMS = pltpu.CompilerParams(use_tc_tiling_on_sc=False)


def _row_block(n):
    return 1000 if n % 1000 == 0 else n


def _film_dense(xin, bn_pack, Wls, Wfsb, Wfsg, w_outs, b_outs, act):
    n, ci = xin.shape
    bn = _row_block(n)
    grid = (n // bn,)
    nouts = len(w_outs)
    has_bias = [b is not None for b in b_outs]

    def body(*refs):
        refs = list(refs)
        if bn_pack is not None:
            yr, csr, cqr, bwr, bbr = refs[:5]
            del refs[:5]
            mu = csr[...] / n
            var = cqr[...] / n - mu * mu
            xb = (yr[...] - mu) * lax.rsqrt(var + _EPS) * bwr[...] + bbr[...]
        else:
            xb = refs.pop(0)[...]
        wls, wfsb, wfsg = refs[0], refs[1], refs[2]
        wrefs = refs[3:3 + nouts]
        brefs = refs[3 + nouts:3 + nouts + sum(has_bias)]
        outs = refs[3 + nouts + sum(has_bias):]

        def dot(wr):
            return lax.dot_general(xb, wr[...], (((1,), (1,)), ((), ())),
                                   preferred_element_type=jnp.float32)

        skip = dot(wfsg) * dot(wls) + dot(wfsb)
        if act:
            skip = jnp.maximum(skip, 0.0)
        outs[0][...] = skip
        bi = 0
        for k in range(nouts):
            r = dot(wrefs[k])
            if has_bias[k]:
                r = r + brefs[bi][...]
                bi += 1
            outs[1 + k][...] = r

    args = [xin]
    in_specs = [pl.BlockSpec((bn, ci), lambda i: (i, 0))]
    if bn_pack is not None:
        for a in bn_pack:
            args.append(a)
            in_specs.append(pl.BlockSpec((1, ci), lambda i: (0, 0)))
    for w in (Wls, Wfsb, Wfsg, *w_outs):
        args.append(w)
        in_specs.append(pl.BlockSpec(w.shape, lambda i: (0, 0)))
    for b in b_outs:
        if b is not None:
            args.append(b)
            in_specs.append(pl.BlockSpec(b.shape, lambda i: (0, 0)))

    co_skip = Wls.shape[0]
    out_shape = [jax.ShapeDtypeStruct((n, co_skip), jnp.float32)]
    out_specs = [pl.BlockSpec((bn, co_skip), lambda i: (i, 0))]
    for w in w_outs:
        out_shape.append(jax.ShapeDtypeStruct((n, w.shape[0]), jnp.float32))
        out_specs.append(pl.BlockSpec((bn, w.shape[0]), lambda i: (i, 0)))

    return pl.pallas_call(
        body, grid=grid, in_specs=in_specs, out_specs=out_specs,
        out_shape=out_shape)(*args)


def _combine(skip, agg, cnt):
    n, co = skip.shape
    bn = _row_block(n)
    grid = (n // bn,)

    def body(sr, ar, cr, yo, cso, cqo):
        c = jnp.maximum(cr[:, 0:1], 1.0)
        agg = jnp.concatenate([ar[0], ar[1]], axis=1)
        y = sr[...] + agg / c
        yo[...] = y

        @pl.when(pl.program_id(0) == 0)
        def _():
            cso[...] = jnp.zeros((1, co), jnp.float32)
            cqo[...] = jnp.zeros((1, co), jnp.float32)

        cso[...] += jnp.sum(y, axis=0, keepdims=True)
        cqo[...] += jnp.sum(y * y, axis=0, keepdims=True)

    return pl.pallas_call(
        body, grid=grid,
        in_specs=[pl.BlockSpec((bn, co), lambda i: (i, 0)),
                  pl.BlockSpec((_CORES, bn, co // 2), lambda i: (0, i, 0)),
                  pl.BlockSpec((bn, _LANES), lambda i: (i, 0))],
        out_specs=[pl.BlockSpec((bn, co), lambda i: (i, 0)),
                   pl.BlockSpec((1, co), lambda i: (0, 0)),
                   pl.BlockSpec((1, co), lambda i: (0, 0))],
        out_shape=[jax.ShapeDtypeStruct((n, co), jnp.float32),
                   jax.ShapeDtypeStruct((1, co), jnp.float32),
                   jax.ShapeDtypeStruct((1, co), jnp.float32)])(skip, agg, cnt)


def _final_combine(skip3, g3, b3, p, cnt):
    n, d = skip3.shape
    bn = _row_block(n)
    grid = (n // bn,)

    def body(sr, gr, br, pr, cr, oo):
        s = pr[0] + pr[1]
        c = cr[...]
        oo[...] = (sr[...] + gr[...] * (s / jnp.maximum(c, 1.0))
                   + br[...] * jnp.minimum(c, 1.0))

    return pl.pallas_call(
        body, grid=grid,
        in_specs=[pl.BlockSpec((bn, d), lambda i: (i, 0)),
                  pl.BlockSpec((bn, d), lambda i: (i, 0)),
                  pl.BlockSpec((bn, d), lambda i: (i, 0)),
                  pl.BlockSpec((_CORES, bn, d), lambda i: (0, i, 0)),
                  pl.BlockSpec((bn, _LANES), lambda i: (i, 0))],
        out_specs=pl.BlockSpec((bn, d), lambda i: (i, 0)),
        out_shape=jax.ShapeDtypeStruct((n, d), jnp.float32))(
            skip3, g3, b3, p, cnt)


def _zero_stripe(buf, dst, row0, rpt):
    nfull = rpt // _CHUNK
    rem = rpt % _CHUNK
    for k in range(nfull):
        pltpu.sync_copy(buf, dst.at[pl.ds(row0 + k * _CHUNK, _CHUNK)])
    if rem:
        pltpu.sync_copy(buf.at[pl.ds(0, rem)],
                        dst.at[pl.ds(row0 + nfull * _CHUNK, rem)])


def _sc_count(dst_p, nacc):
    ep = dst_p.shape[0]
    dst2 = dst_p.reshape(ep // _CHUNK, _CHUNK)
    rows_per_tile = (ep // _CHUNK) // _TILES
    nblocks = rows_per_tile // _IDXB
    rpt = nacc // _TILES

    mesh = plsc.VectorSubcoreMesh(core_axis_name="c", subcore_axis_name="s")

    def body(dstr, cnto, didx, ones, cacc, sem):
        cid = lax.axis_index("c")
        sid = lax.axis_index("s")
        row0 = sid * rpt

        @pl.when(cid == 0)
        def _():
            @pl.loop(0, _CHUNK)
            def _(e):
                ones[pl.ds(e, 1), pl.ds(0, _LANES)] = jnp.zeros(
                    (1, _LANES), jnp.float32)

            _zero_stripe(ones, cacc, row0, rpt)

            @pl.loop(0, _CHUNK)
            def _(e):
                ones[pl.ds(e, 1), pl.ds(0, _LANES)] = jnp.ones(
                    (1, _LANES), jnp.float32)

            plsc.subcore_barrier()

            @pl.loop(0, nblocks)
            def _(t):
                blk = sid * rows_per_tile + t * _IDXB
                pltpu.sync_copy(dstr.at[pl.ds(blk, _IDXB)], didx)

                @pl.loop(0, _IDXB)
                def _(j):
                    pltpu.sync_copy(ones, cacc.at[didx.at[j]], add=True)

            plsc.subcore_barrier()
            pltpu.sync_copy(cacc.at[pl.ds(row0, rpt)],
                            cnto.at[pl.ds(row0, rpt)])

    scratch = [
        pltpu.VMEM((_IDXB, _CHUNK), jnp.int32),
        pltpu.VMEM((_CHUNK, _LANES), jnp.float32),
        pltpu.VMEM_SHARED((nacc, _LANES), jnp.float32),
        pltpu.SemaphoreType.DMA,
    ]
    fn = pl.kernel(body,
                   out_type=jax.ShapeDtypeStruct((nacc, _LANES), jnp.float32),
                   mesh=mesh, scratch_types=scratch,
                   compiler_params=_SC_PARAMS)
    return fn(dst2)


def _sc_edge(xw_lo, xw_hi, g_lo, g_hi, b_lo, b_hi, src_p, dst_p, nacc):
    ep = src_p.shape[0]
    src2 = src_p.reshape(ep // _CHUNK, _CHUNK)
    dst2 = dst_p.reshape(ep // _CHUNK, _CHUNK)
    nrows = ep // _CHUNK
    rows_per_tile = nrows // _TILES
    nblocks = rows_per_tile // _IDXB
    rpt = nacc // _TILES
    half = xw_lo.shape[1]
    nj = half // _LANES

    mesh = plsc.VectorSubcoreMesh(core_axis_name="c", subcore_axis_name="s")

    def body(xl, xh, gl, gh, bl, bh, srcr, dstr, aggo,
             sidx, didx, bufx, bufg, bufb, acc, sem):
        cid = lax.axis_index("c")
        sid = lax.axis_index("s")
        row0 = sid * rpt

        @pl.loop(0, _CHUNK)
        def _(e):
            for j in range(nj):
                bufx[pl.ds(e, 1), pl.ds(j * _LANES, _LANES)] = jnp.zeros(
                    (1, _LANES), jnp.float32)

        _zero_stripe(bufx, acc, row0, rpt)
        plsc.subcore_barrier()

        def run_half(xw, g, b):
            rbase = sid * rows_per_tile

            @pl.loop(0, nblocks)
            def _(t):
                blk = rbase + t * _IDXB
                pltpu.sync_copy(srcr.at[pl.ds(blk, _IDXB)], sidx)
                pltpu.sync_copy(dstr.at[pl.ds(blk, _IDXB)], didx)

                @pl.loop(0, _IDXB)
                def _(j):
                    cx = pltpu.async_copy(xw.at[sidx.at[j]], bufx, sem)
                    cg = pltpu.async_copy(g.at[didx.at[j]], bufg, sem)
                    cb = pltpu.async_copy(b.at[didx.at[j]], bufb, sem)
                    cx.wait()
                    cg.wait()
                    cb.wait()

                    @pl.loop(0, _CHUNK)
                    def _(e):
                        for k in range(nj):
                            sl = (pl.ds(e, 1), pl.ds(k * _LANES, _LANES))
                            bufx[sl] = jnp.maximum(
                                bufg[sl] * bufx[sl] + bufb[sl], 0.0)

                    pltpu.sync_copy(bufx, acc.at[didx.at[j]], add=True)

        @pl.when(cid == 0)
        def _():
            run_half(xl, gl, bl)

        @pl.when(cid == 1)
        def _():
            run_half(xh, gh, bh)

        plsc.subcore_barrier()

        @pl.when(cid == 0)
        def _():
            pltpu.sync_copy(acc.at[pl.ds(row0, rpt)],
                            aggo.at[0, pl.ds(row0, rpt)])

        @pl.when(cid == 1)
        def _():
            pltpu.sync_copy(acc.at[pl.ds(row0, rpt)],
                            aggo.at[1, pl.ds(row0, rpt)])

    scratch = [
        pltpu.VMEM((_IDXB, _CHUNK), jnp.int32),
        pltpu.VMEM((_IDXB, _CHUNK), jnp.int32),
        pltpu.VMEM((_CHUNK, half), jnp.float32),
        pltpu.VMEM((_CHUNK, half), jnp.float32),
        pltpu.VMEM((_CHUNK, half), jnp.float32),
        pltpu.VMEM_SHARED((nacc, half), jnp.float32),
        pltpu.SemaphoreType.DMA,
    ]
    fn = pl.kernel(body,
                   out_type=jax.ShapeDtypeStruct((_CORES, nacc, half),
                                                 jnp.float32),
                   mesh=mesh, scratch_types=scratch,
                   compiler_params=_SC_PARAMS)
    return fn(xw_lo, xw_hi, g_lo, g_hi, b_lo, b_hi, src2, dst2)


def _sc_final(xw3, src_p, dst_p, nacc):
    ep = src_p.shape[0]
    src2 = src_p.reshape(ep // _CHUNK, _CHUNK)
    dst2 = dst_p.reshape(ep // _CHUNK, _CHUNK)
    rows_per_w = (ep // _CHUNK) // (_TILES * _CORES)
    nb = _IDXB if rows_per_w % _IDXB == 0 else (
        4 if rows_per_w % 4 == 0 else 1)
    nblocks = rows_per_w // nb
    rpt = nacc // _TILES
    d = xw3.shape[1]

    mesh = plsc.VectorSubcoreMesh(core_axis_name="c", subcore_axis_name="s")

    def body(xwr, srcr, dstr, po, sidx, didx, bufx, acc, sem):
        cid = lax.axis_index("c")
        sid = lax.axis_index("s")
        row0 = sid * rpt

        @pl.loop(0, _CHUNK)
        def _(e):
            bufx[pl.ds(e, 1), pl.ds(0, d)] = jnp.zeros((1, d), jnp.float32)

        _zero_stripe(bufx, acc, row0, rpt)
        plsc.subcore_barrier()

        wbase = (cid * _TILES + sid) * rows_per_w

        @pl.loop(0, nblocks)
        def _(t):
            blk = wbase + t * nb
            pltpu.sync_copy(srcr.at[pl.ds(blk, nb)], sidx)
            pltpu.sync_copy(dstr.at[pl.ds(blk, nb)], didx)

            @pl.loop(0, nb)
            def _(j):
                pltpu.async_copy(xwr.at[sidx.at[j]], bufx, sem).wait()
                pltpu.sync_copy(bufx, acc.at[didx.at[j]], add=True)

        plsc.subcore_barrier()

        @pl.when(cid == 0)
        def _():
            pltpu.sync_copy(acc.at[pl.ds(row0, rpt)],
                            po.at[0, pl.ds(row0, rpt)])

        @pl.when(cid == 1)
        def _():
            pltpu.sync_copy(acc.at[pl.ds(row0, rpt)],
                            po.at[1, pl.ds(row0, rpt)])

    scratch = [
        pltpu.VMEM((nb, _CHUNK), jnp.int32),
        pltpu.VMEM((nb, _CHUNK), jnp.int32),
        pltpu.VMEM((_CHUNK, d), jnp.float32),
        pltpu.VMEM_SHARED((nacc, d), jnp.float32),
        pltpu.SemaphoreType.DMA,
    ]
    fn = pl.kernel(body, out_type=jax.ShapeDtypeStruct((_CORES, nacc, d),
                                                       jnp.float32),
                   mesh=mesh, scratch_types=scratch,
                   compiler_params=_SC_PARAMS)
    return fn(xw3, src2, dst2)


def _split_weights(W_lin, W_film, b_film, W_fs, split):
    co = W_lin.shape[0]
    Wfsb, Wfsg = W_fs[:co], W_fs[co:]
    Wfmb, Wfmg = W_film[:co], W_film[co:]
    bfb = b_film[:co].reshape(1, co)
    bfg = b_film[co:].reshape(1, co)
    if not split:
        return Wfsb, Wfsg, [Wfmb, Wfmg, W_lin], [bfb, bfg, None]
    h = co // 2
    w_outs = [Wfmb[:h], Wfmb[h:], Wfmg[:h], Wfmg[h:], W_lin[:h], W_lin[h:]]
    b_outs = [bfb[:, :h], bfb[:, h:], bfg[:, :h], bfg[:, h:], None, None]
    return Wfsb, Wfsg, w_outs, b_outs


def kernel(x, edge_index,
           W_lin0, W_film0, b_film0, W_ls0, W_fs0,
           W_lin1, W_film1, b_film1, W_ls1, W_fs1,
           W_lin2, W_film2, b_film2, W_ls2, W_fs2,
           W_lin3, W_film3, b_film3, W_ls3, W_fs3,
           bn_w0, bn_b0, bn_w1, bn_b1, bn_w2, bn_b2):
    n = x.shape[0]
    e = edge_index.shape[1]
    nacc = -(-(n + 1) // _TILES) * _TILES
    epad = _TILES * _IDXB * _CHUNK
    ep = -(-e // epad) * epad
    src_p = jnp.concatenate(
        [edge_index[0], jnp.zeros((ep - e,), edge_index.dtype)])
    dst_p = jnp.concatenate(
        [edge_index[1], jnp.full((ep - e,), nacc - 1, edge_index.dtype)])

    cnt = _sc_count(dst_p, nacc)

    layer_w = [(W_lin0, W_film0, b_film0, W_ls0, W_fs0),
               (W_lin1, W_film1, b_film1, W_ls1, W_fs1),
               (W_lin2, W_film2, b_film2, W_ls2, W_fs2)]
    bn_params = [(bn_w0, bn_b0), (bn_w1, bn_b1), (bn_w2, bn_b2)]

    h = x
    bn_pack = None
    for i in range(3):
        W_lin, W_film, b_film, W_ls, W_fs = layer_w[i]
        Wfsb, Wfsg, w_outs, b_outs = _split_weights(W_lin, W_film, b_film,
                                                    W_fs, split=True)
        skip, bl, bh, gl, gh, xl, xh = _film_dense(
            h, bn_pack, W_ls, Wfsb, Wfsg, w_outs, b_outs, act=True)
        padrows = ((0, nacc - n), (0, 0))
        agg = _sc_edge(xl, xh,
                       jnp.pad(gl, padrows), jnp.pad(gh, padrows),
                       jnp.pad(bl, padrows), jnp.pad(bh, padrows),
                       src_p, dst_p, nacc)
        y, cs, cq = _combine(skip, agg, cnt)
        bw = bn_params[i][0].reshape(1, y.shape[1])
        bb = bn_params[i][1].reshape(1, y.shape[1])
        bn_pack = (cs, cq, bw, bb)
        h = y

    Wfsb3, Wfsg3, w_outs3, b_outs3 = _split_weights(
        W_lin3, W_film3, b_film3, W_fs3, split=False)
    skip3, b3, g3, xw3 = _film_dense(
        h, bn_pack, W_ls3, Wfsb3, Wfsg3, w_outs3, b_outs3, act=False)
    p = _sc_final(xw3, src_p, dst_p, nacc)
    return _final_combine(skip3, g3, b3, p, cnt)
